# baseline (speedup 1.0000x reference)
import functools

import jax
import jax.numpy as jnp
from jax import lax
from jax.experimental import pallas as pl
from jax.experimental.pallas import tpu as pltpu
from jax.experimental.pallas import tpu_sc as plsc

N = 10000
E = 320000
HID = 64

NC = 2
NS = 16
CW = 80
NCHUNK = E // CW
RPT = N // NS
_NB = 5

_MESH = plsc.VectorSubcoreMesh(
    core_axis_name="c", subcore_axis_name="s", num_cores=NC, num_subcores=NS)
_SC_PARAMS = pltpu.CompilerParams(use_tc_tiling_on_sc=False)


def _fill(buf, nrows, d, value):
    @pl.loop(0, nrows)
    def _(i):
        for k in range(d // 16):
            buf[i, pl.ds(k * 16, 16)] = jnp.full((16,), value, jnp.float32)


def _zero_acc(zbuf, acc, s):
    base = s * RPT
    for r in range(RPT // CW):
        pltpu.sync_copy(zbuf, acc.at[pl.ds(base + r * CW, CW)])
    rem = RPT - (RPT // CW) * CW
    pltpu.sync_copy(zbuf.at[pl.ds(0, rem)],
                    acc.at[pl.ds(base + RPT - rem, rem)])


def _idx_scale(idx, cpt, mult, off):
    @pl.loop(0, cpt)
    def _(i):
        for k in range(CW // 16):
            sl = pl.ds(k * 16, 16)
            idx[i, sl] = idx[i, sl] * mult + off


DCW = 125
DNCHUNK = E // DCW


def _make_deg():
    cpt = DNCHUNK // (NC * NS)

    @functools.partial(
        pl.kernel,
        out_type=[jax.ShapeDtypeStruct((N, 128), jnp.float32)],
        mesh=_MESH,
        compiler_params=_SC_PARAMS,
        scratch_types=[
            pltpu.VMEM((cpt, DCW), jnp.int32),
            pltpu.VMEM((DCW, 16), jnp.float32),
            pltpu.VMEM((DCW, 16), jnp.float32),
            pltpu.VMEM_SHARED((N, 16), jnp.float32),
        ],
    )
    def deg_kernel(er, out, didx, ones, zeros, acc):
        c = lax.axis_index("c")
        s = lax.axis_index("s")

        _fill(ones, DCW, 16, 1.0)
        _fill(zeros, DCW, 16, 0.0)
        base = s * RPT
        for r in range(RPT // DCW):
            pltpu.sync_copy(zeros, acc.at[pl.ds(base + r * DCW, DCW)])
        plsc.subcore_barrier()

        wid = c * NS + s
        pltpu.sync_copy(er.at[1, pl.ds(wid * cpt, cpt)], didx)

        @pl.loop(0, cpt)
        def _(j):
            pltpu.sync_copy(ones, acc.at[didx.at[j]], add=True)

        plsc.subcore_barrier()
        pltpu.sync_copy(acc.at[pl.ds(s * RPT, RPT)],
                        out.at[pl.ds(s * RPT, RPT), pl.ds(c * 16, 16)])

    return deg_kernel


def _make_prop(d, col_split, mult, off):
    cpt = NCHUNK // NS if col_split else NCHUNK // (NC * NS)

    @functools.partial(
        pl.kernel,
        out_type=[jax.ShapeDtypeStruct((N, 128), jnp.float32)],
        mesh=_MESH,
        compiler_params=_SC_PARAMS,
        scratch_types=[
            pltpu.VMEM((cpt, CW), jnp.int32),
            pltpu.VMEM((cpt, CW), jnp.int32),
            [pltpu.VMEM((CW, d), jnp.float32) for _ in range(_NB)],
            pltpu.VMEM_SHARED((N, d), jnp.float32),
            pltpu.SemaphoreType.DMA,
            pltpu.SemaphoreType.DMA,
        ],
    )
    def prop_kernel(in_hbm, er, out, sidx, didx, bufs, acc, gsem, ssem):
        c = lax.axis_index("c")
        s = lax.axis_index("s")

        _fill(bufs[0], CW, d, 0.0)
        _zero_acc(bufs[0], acc, s)
        plsc.subcore_barrier()

        base = (s if col_split else c * NS + s) * cpt
        pltpu.sync_copy(er.at[0, pl.ds(base, cpt)], sidx)
        pltpu.sync_copy(er.at[1, pl.ds(base, cpt)], didx)
        _idx_scale(sidx, cpt, mult, c if off is None else off)

        for b in range(_NB - 1):
            pltpu.async_copy(in_hbm.at[sidx.at[b]], bufs[b], gsem)

        @pl.loop(0, cpt, step=_NB)
        def _(j):
            for b in range(_NB):
                jj = j + b
                buf = bufs[b]
                pltpu.make_async_copy(in_hbm.at[sidx.at[jj]], buf, gsem).wait()
                pltpu.async_copy(buf, acc.at[didx.at[jj]], ssem, add=True)

                @pl.when(jj + (_NB - 1) < cpt)
                def _():
                    nbuf = bufs[(b + _NB - 1) % _NB]

                    @pl.when(jj >= 1)
                    def _():
                        pltpu.make_async_copy(
                            nbuf, acc.at[didx.at[jj]], ssem).wait()

                    pltpu.async_copy(
                        in_hbm.at[sidx.at[jj + (_NB - 1)]], nbuf, gsem)

        for b in range(_NB):
            pltpu.make_async_copy(bufs[b], acc.at[didx.at[0]], ssem).wait()

        plsc.subcore_barrier()
        pltpu.sync_copy(acc.at[pl.ds(s * RPT, RPT)],
                        out.at[pl.ds(s * RPT, RPT), pl.ds(c * d, d)])

    return prop_kernel


def _make_prop_fold(d, mult, add_col, scale_cols, out_cols):
    cpt = NCHUNK // NS

    @functools.partial(
        pl.kernel,
        out_type=[jax.ShapeDtypeStruct((N, 128), jnp.float32)],
        mesh=_MESH,
        compiler_params=_SC_PARAMS,
        scratch_types=[
            pltpu.VMEM((cpt, CW), jnp.int32),
            pltpu.VMEM((cpt, CW), jnp.int32),
            [pltpu.VMEM((CW, d), jnp.float32) for _ in range(_NB)],
            pltpu.VMEM((125, d), jnp.float32),
            pltpu.VMEM((125, 16), jnp.float32),
            pltpu.VMEM((125, d), jnp.float32),
            pltpu.VMEM_SHARED((N, d), jnp.float32),
            pltpu.SemaphoreType.DMA,
            pltpu.SemaphoreType.DMA,
        ],
    )
    def prop_kernel(in_hbm, er, aux, out,
                    sidx, didx, bufs, dbuf, sbuf, abuf, acc, gsem, ssem):
        c = lax.axis_index("c")
        s = lax.axis_index("s")

        _fill(bufs[0], CW, d, 0.0)
        _zero_acc(bufs[0], acc, s)
        plsc.subcore_barrier()

        base = s * cpt
        pltpu.sync_copy(er.at[0, pl.ds(base, cpt)], sidx)
        pltpu.sync_copy(er.at[1, pl.ds(base, cpt)], didx)
        _idx_scale(sidx, cpt, mult, c)

        for b in range(_NB - 1):
            pltpu.async_copy(in_hbm.at[sidx.at[b]], bufs[b], gsem)

        @pl.loop(0, cpt, step=_NB)
        def _(j):
            for b in range(_NB):
                jj = j + b
                buf = bufs[b]
                pltpu.make_async_copy(in_hbm.at[sidx.at[jj]], buf, gsem).wait()
                pltpu.async_copy(buf, acc.at[didx.at[jj]], ssem, add=True)

                @pl.when(jj + (_NB - 1) < cpt)
                def _():
                    nbuf = bufs[(b + _NB - 1) % _NB]

                    @pl.when(jj >= 1)
                    def _():
                        pltpu.make_async_copy(
                            nbuf, acc.at[didx.at[jj]], ssem).wait()

                    pltpu.async_copy(
                        in_hbm.at[sidx.at[jj + (_NB - 1)]], nbuf, gsem)

        for b in range(_NB):
            pltpu.make_async_copy(bufs[b], acc.at[didx.at[0]], ssem).wait()

        plsc.subcore_barrier()

        def drain(scale_col, out_col, with_add):
            for r in range(RPT // 125):
                a0 = s * RPT + r * 125
                rows = pl.ds(a0, 125)
                pltpu.sync_copy(acc.at[pl.ds(a0, 125)], dbuf)
                pltpu.sync_copy(aux.at[rows, pl.ds(scale_col, 16)], sbuf)
                if with_add:
                    pltpu.sync_copy(aux.at[rows, pl.ds(add_col, d)], abuf)

                @pl.loop(0, 125)
                def _(i):
                    for k in range(d // 16):
                        sl = pl.ds(k * 16, 16)
                        v = dbuf[i, sl] * sbuf[i, pl.ds(0, 16)]
                        if with_add:
                            v = v + abuf[i, sl]
                        dbuf[i, sl] = v

                pltpu.sync_copy(dbuf, out.at[rows, pl.ds(out_col, d)])

        @pl.when(c == 0)
        def _():
            drain(scale_cols[0], out_cols[0], True)

        @pl.when(c == 1)
        def _():
            drain(scale_cols[1], out_cols[1], False)

    return prop_kernel


_deg = _make_deg()
_prop_s1 = _make_prop(64, True, 2, None)
_prop_s2 = _make_prop(64, False, 2, 1)
_prop_s3 = _make_prop(64, False, 2, 0)
_prop_s4 = _make_prop(32, False, 4, 0)


def _dot(a, b):
    return jnp.dot(a, b, preferred_element_type=jnp.float32,
                   precision=jax.lax.Precision.HIGHEST)


def _t1_body(x_ref, w0_ref, w1_ref, w2_ref, b_ref, dp_ref,
             aux_ref, a12_ref):
    deg = jnp.maximum(dp_ref[:, 0] + dp_ref[:, 16], 1.0)
    isd = jax.lax.rsqrt(deg)[:, None]
    x = x_ref[...]
    out0b = _dot(x, w0_ref[...]) + b_ref[...]
    xs = x * isd
    w12 = jnp.concatenate([w1_ref[...], w2_ref[...]], axis=1)
    a12_ref[...] = _dot(xs, w12)
    isd2 = isd * isd
    pad = jnp.zeros((isd.shape[0], 62), jnp.float32)
    aux_ref[...] = jnp.concatenate([out0b, isd, isd2, pad], axis=1)


def _t2_body(aux_ref, u12_ref, m2_ref):
    u12 = u12_ref[...]
    acc1 = aux_ref[:, :HID] + aux_ref[:, HID:HID + 1] * u12[:, :HID]
    qs = aux_ref[:, HID + 1:HID + 2] * u12[:, HID:]
    m2_ref[...] = jnp.concatenate([acc1, qs], axis=1)


def _t3_body(m2_ref, q_ref, aux_ref, w0_ref, w1_ref, w2_ref, b_ref, m3_ref):
    isd = aux_ref[:, HID:HID + 1]
    q = q_ref[...]
    h1 = jax.nn.relu(m2_ref[:, :HID] + isd * (q[:, :HID] + q[:, HID:]))
    out0b2 = _dot(h1, w0_ref[...]) + b_ref[...]
    w12 = jnp.concatenate([w1_ref[...], w2_ref[...]], axis=1)
    bs = _dot(h1 * isd, w12)
    m3_ref[...] = jnp.concatenate(
        [bs, out0b2, jnp.zeros_like(out0b2)], axis=1)


def _t4_body(m3_ref, v_ref, aux_ref, m4_ref):
    vv = v_ref[...]
    v = vv[:, :HID] + vv[:, HID:]
    acc2 = m3_ref[:, HID:HID + 32] + aux_ref[:, HID:HID + 1] * v[:, :32]
    rs = aux_ref[:, HID + 1:HID + 2] * v[:, 32:]
    m4_ref[...] = jnp.concatenate(
        [rs, acc2, jnp.zeros((rs.shape[0], HID), jnp.float32)], axis=1)


def _t5_body(m4_ref, w_ref, aux_ref,
             cw1_ref, cb1_ref, cw2_ref, cb2_ref, coords_ref):
    wv = w_ref[:, :32] + w_ref[:, 32:64]
    h2 = jax.nn.relu(m4_ref[:, 32:64] + aux_ref[:, HID:HID + 1] * wv)
    t = jax.nn.relu(jnp.dot(h2, cw1_ref[...],
                            preferred_element_type=jnp.float32) + cb1_ref[...])
    coords_ref[...] = jnp.dot(t, cw2_ref[...],
                              preferred_element_type=jnp.float32) + cb2_ref[...]


_BN = 1000


def _tc_call(body, out_shapes, *args):
    in_specs, arrays = [], []
    for a in args:
        if isinstance(a, tuple):
            a, (ncols, cb) = a
            in_specs.append(pl.BlockSpec((_BN, ncols), lambda i, cb=cb: (i, cb)))
        elif a.shape[0] == N:
            in_specs.append(pl.BlockSpec(
                (_BN,) + a.shape[1:],
                lambda i, nd=a.ndim: (i,) + (0,) * (nd - 1)))
        else:
            in_specs.append(pl.BlockSpec(
                a.shape, lambda i, nd=a.ndim: (0,) * nd))
        arrays.append(a)
    out_specs = [
        pl.BlockSpec((_BN,) + s[1:], lambda i, nd=len(s): (i,) + (0,) * (nd - 1))
        for s in out_shapes
    ]
    return pl.pallas_call(
        body,
        grid=(N // _BN,),
        in_specs=in_specs,
        out_specs=out_specs,
        out_shape=[jax.ShapeDtypeStruct(s, jnp.float32) for s in out_shapes],
    )(*arrays)


def kernel(x, conv1_w0, conv1_w1, conv1_w2, conv1_b,
           conv2_w0, conv2_w1, conv2_w2, conv2_b,
           proj_w1, proj_b1, proj_w2, proj_b2,
           coord_w1, coord_b1, coord_w2, coord_b2,
           edge_index):
    er = edge_index.reshape(2, NCHUNK, CW)

    (dp,) = _deg(edge_index.reshape(2, DNCHUNK, DCW))

    aux, a12 = _tc_call(
        _t1_body, [(N, 128), (N, 128)],
        x, conv1_w0, conv1_w1, conv1_w2, conv1_b.reshape(1, HID), dp)

    (u12,) = _prop_s1(a12.reshape(2 * N, HID), er)

    (m2,) = _tc_call(_t2_body, [(N, 128)], aux, u12)

    (q,) = _prop_s2(m2.reshape(2 * N, HID), er)

    (m3,) = _tc_call(
        _t3_body, [(N, 128)],
        m2, q, aux, conv2_w0, conv2_w1, conv2_w2, conv2_b.reshape(1, 32))

    (v,) = _prop_s3(m3.reshape(2 * N, HID), er)

    (m4,) = _tc_call(_t4_body, [(N, 128)], m3, v, aux)

    (w,) = _prop_s4(m4.reshape(4 * N, 32), er)

    (coords,) = _tc_call(
        _t5_body, [(N, 2)],
        m4, w, aux,
        coord_w1, coord_b1.reshape(1, 16), coord_w2, coord_b2.reshape(1, 2))

    return coords


# speedup vs baseline: 1.0683x; 1.0683x over previous
import functools

import jax
import jax.numpy as jnp
from jax import lax
from jax.experimental import pallas as pl
from jax.experimental.pallas import tpu as pltpu
from jax.experimental.pallas import tpu_sc as plsc

N = 10000
E = 320000
HID = 64

NC = 2
NS = 16
CW = 80
NCHUNK = E // CW
RPT = N // NS
_NB = 5

_MESH = plsc.VectorSubcoreMesh(
    core_axis_name="c", subcore_axis_name="s", num_cores=NC, num_subcores=NS)
_SC_PARAMS = pltpu.CompilerParams(use_tc_tiling_on_sc=False)


def _fill(buf, nrows, d, value):
    @pl.loop(0, nrows)
    def _(i):
        for k in range(d // 16):
            buf[i, pl.ds(k * 16, 16)] = jnp.full((16,), value, jnp.float32)


def _zero_acc(zbuf, acc, s):
    base = s * RPT
    for r in range(RPT // CW):
        pltpu.sync_copy(zbuf, acc.at[pl.ds(base + r * CW, CW)])
    rem = RPT - (RPT // CW) * CW
    pltpu.sync_copy(zbuf.at[pl.ds(0, rem)],
                    acc.at[pl.ds(base + RPT - rem, rem)])


def _idx_scale(idx, cpt, mult, off):
    @pl.loop(0, cpt)
    def _(i):
        for k in range(CW // 16):
            sl = pl.ds(k * 16, 16)
            idx[i, sl] = idx[i, sl] * mult + off


DCW = 125
DNCHUNK = E // DCW


def _make_deg():
    cpt = DNCHUNK // (NC * NS)

    @functools.partial(
        pl.kernel,
        out_type=[jax.ShapeDtypeStruct((N, 128), jnp.float32)],
        mesh=_MESH,
        compiler_params=_SC_PARAMS,
        scratch_types=[
            pltpu.VMEM((cpt, DCW), jnp.int32),
            pltpu.VMEM((DCW, 16), jnp.float32),
            pltpu.VMEM((DCW, 16), jnp.float32),
            pltpu.VMEM_SHARED((N, 16), jnp.float32),
        ],
    )
    def deg_kernel(er, out, didx, ones, zeros, acc):
        c = lax.axis_index("c")
        s = lax.axis_index("s")

        _fill(ones, DCW, 16, 1.0)
        _fill(zeros, DCW, 16, 0.0)
        base = s * RPT
        for r in range(RPT // DCW):
            pltpu.sync_copy(zeros, acc.at[pl.ds(base + r * DCW, DCW)])
        plsc.subcore_barrier()

        wid = c * NS + s
        pltpu.sync_copy(er.at[1, pl.ds(wid * cpt, cpt)], didx)

        @pl.loop(0, cpt)
        def _(j):
            pltpu.sync_copy(ones, acc.at[didx.at[j]], add=True)

        plsc.subcore_barrier()
        pltpu.sync_copy(acc.at[pl.ds(s * RPT, RPT)],
                        out.at[pl.ds(s * RPT, RPT), pl.ds(c * 16, 16)])

    return deg_kernel


def _make_prop(d, col_split, mult, off):
    cpt = NCHUNK // NS if col_split else NCHUNK // (NC * NS)

    @functools.partial(
        pl.kernel,
        out_type=[jax.ShapeDtypeStruct((N, 128), jnp.float32)],
        mesh=_MESH,
        compiler_params=_SC_PARAMS,
        scratch_types=[
            pltpu.VMEM((cpt, CW), jnp.int32),
            pltpu.VMEM((cpt, CW), jnp.int32),
            [pltpu.VMEM((CW, d), jnp.float32) for _ in range(_NB)],
            pltpu.VMEM_SHARED((N, d), jnp.float32),
            pltpu.SemaphoreType.DMA,
            pltpu.SemaphoreType.DMA,
        ],
    )
    def prop_kernel(in_hbm, er, out, sidx, didx, bufs, acc, gsem, ssem):
        c = lax.axis_index("c")
        s = lax.axis_index("s")

        _fill(bufs[0], CW, d, 0.0)
        _zero_acc(bufs[0], acc, s)
        plsc.subcore_barrier()

        base = (s if col_split else c * NS + s) * cpt
        pltpu.sync_copy(er.at[0, pl.ds(base, cpt)], sidx)
        pltpu.sync_copy(er.at[1, pl.ds(base, cpt)], didx)
        _idx_scale(sidx, cpt, mult, c if off is None else off)

        for b in range(_NB - 1):
            pltpu.async_copy(in_hbm.at[sidx.at[b]], bufs[b], gsem)

        @pl.loop(0, cpt, step=_NB)
        def _(j):
            for b in range(_NB):
                jj = j + b
                buf = bufs[b]
                pltpu.make_async_copy(in_hbm.at[sidx.at[jj]], buf, gsem).wait()
                pltpu.async_copy(buf, acc.at[didx.at[jj]], ssem, add=True)

                @pl.when(jj + (_NB - 1) < cpt)
                def _():
                    nbuf = bufs[(b + _NB - 1) % _NB]

                    @pl.when(jj >= 1)
                    def _():
                        pltpu.make_async_copy(
                            nbuf, acc.at[didx.at[jj]], ssem).wait()

                    pltpu.async_copy(
                        in_hbm.at[sidx.at[jj + (_NB - 1)]], nbuf, gsem)

        for b in range(_NB):
            pltpu.make_async_copy(bufs[b], acc.at[didx.at[0]], ssem).wait()

        plsc.subcore_barrier()
        pltpu.sync_copy(acc.at[pl.ds(s * RPT, RPT)],
                        out.at[pl.ds(s * RPT, RPT), pl.ds(c * d, d)])

    return prop_kernel


def _make_prop_fold(d, mult, add_col, scale_cols, out_cols):
    cpt = NCHUNK // NS

    @functools.partial(
        pl.kernel,
        out_type=[jax.ShapeDtypeStruct((N, 128), jnp.float32)],
        mesh=_MESH,
        compiler_params=_SC_PARAMS,
        scratch_types=[
            pltpu.VMEM((cpt, CW), jnp.int32),
            pltpu.VMEM((cpt, CW), jnp.int32),
            [pltpu.VMEM((CW, d), jnp.float32) for _ in range(_NB)],
            pltpu.VMEM((125, d), jnp.float32),
            pltpu.VMEM((125, 16), jnp.float32),
            pltpu.VMEM((125, d), jnp.float32),
            pltpu.VMEM_SHARED((N, d), jnp.float32),
            pltpu.SemaphoreType.DMA,
            pltpu.SemaphoreType.DMA,
        ],
    )
    def prop_kernel(in_hbm, er, aux, out,
                    sidx, didx, bufs, dbuf, sbuf, abuf, acc, gsem, ssem):
        c = lax.axis_index("c")
        s = lax.axis_index("s")

        _fill(bufs[0], CW, d, 0.0)
        _zero_acc(bufs[0], acc, s)
        plsc.subcore_barrier()

        base = s * cpt
        pltpu.sync_copy(er.at[0, pl.ds(base, cpt)], sidx)
        pltpu.sync_copy(er.at[1, pl.ds(base, cpt)], didx)
        _idx_scale(sidx, cpt, mult, c)

        for b in range(_NB - 1):
            pltpu.async_copy(in_hbm.at[sidx.at[b]], bufs[b], gsem)

        @pl.loop(0, cpt, step=_NB)
        def _(j):
            for b in range(_NB):
                jj = j + b
                buf = bufs[b]
                pltpu.make_async_copy(in_hbm.at[sidx.at[jj]], buf, gsem).wait()
                pltpu.async_copy(buf, acc.at[didx.at[jj]], ssem, add=True)

                @pl.when(jj + (_NB - 1) < cpt)
                def _():
                    nbuf = bufs[(b + _NB - 1) % _NB]

                    @pl.when(jj >= 1)
                    def _():
                        pltpu.make_async_copy(
                            nbuf, acc.at[didx.at[jj]], ssem).wait()

                    pltpu.async_copy(
                        in_hbm.at[sidx.at[jj + (_NB - 1)]], nbuf, gsem)

        for b in range(_NB):
            pltpu.make_async_copy(bufs[b], acc.at[didx.at[0]], ssem).wait()

        plsc.subcore_barrier()

        def drain(scale_col, out_col, with_add):
            for r in range(RPT // 125):
                a0 = s * RPT + r * 125
                rows = pl.ds(a0, 125)
                pltpu.sync_copy(acc.at[pl.ds(a0, 125)], dbuf)
                pltpu.sync_copy(aux.at[rows, pl.ds(scale_col, 16)], sbuf)
                if with_add:
                    pltpu.sync_copy(aux.at[rows, pl.ds(add_col, d)], abuf)

                @pl.loop(0, 125)
                def _(i):
                    for k in range(d // 16):
                        sl = pl.ds(k * 16, 16)
                        v = dbuf[i, sl] * sbuf[i, pl.ds(0, 16)]
                        if with_add:
                            v = v + abuf[i, sl]
                        dbuf[i, sl] = v

                pltpu.sync_copy(dbuf, out.at[rows, pl.ds(out_col, d)])

        @pl.when(c == 0)
        def _():
            drain(scale_cols[0], out_cols[0], True)

        @pl.when(c == 1)
        def _():
            drain(scale_cols[1], out_cols[1], False)

    return prop_kernel


_deg = _make_deg()
_prop_s1 = _make_prop(64, True, 2, None)
_prop_s2 = _make_prop(64, False, 2, 1)
_prop_s3 = _make_prop(64, False, 2, 0)
_prop_s4 = _make_prop(32, False, 4, 0)


def _dot(a, b):
    return jnp.dot(a, b, preferred_element_type=jnp.float32,
                   precision=jax.lax.Precision.HIGHEST)


def _t1_body(x_ref, w0_ref, w1_ref, w2_ref, b_ref, dp_ref,
             aux_ref, a12_ref):
    deg = jnp.maximum(dp_ref[:, 0] + dp_ref[:, 16], 1.0)
    isd = jax.lax.rsqrt(deg)[:, None]
    x = x_ref[...]
    out0b = _dot(x, w0_ref[...]) + b_ref[...]
    xs = x * isd
    w12 = jnp.concatenate([w1_ref[...], w2_ref[...]], axis=1)
    a12_ref[...] = _dot(xs, w12)
    isd2 = isd * isd
    pad = jnp.zeros((isd.shape[0], 62), jnp.float32)
    aux_ref[...] = jnp.concatenate([out0b, isd, isd2, pad], axis=1)


def _t2_body(aux_ref, u12_ref, m2_ref):
    u12 = u12_ref[...]
    acc1 = aux_ref[:, :HID] + aux_ref[:, HID:HID + 1] * u12[:, :HID]
    qs = aux_ref[:, HID + 1:HID + 2] * u12[:, HID:]
    m2_ref[...] = jnp.concatenate([acc1, qs], axis=1)


def _t3_body(m2_ref, q_ref, aux_ref, w0_ref, w1_ref, w2_ref, b_ref, m3_ref):
    isd = aux_ref[:, HID:HID + 1]
    q = q_ref[...]
    h1 = jax.nn.relu(m2_ref[:, :HID] + isd * (q[:, :HID] + q[:, HID:]))
    out0b2 = _dot(h1, w0_ref[...]) + b_ref[...]
    w12 = jnp.concatenate([w1_ref[...], w2_ref[...]], axis=1)
    bs = _dot(h1 * isd, w12)
    m3_ref[...] = jnp.concatenate(
        [bs, out0b2, jnp.zeros_like(out0b2)], axis=1)


def _t4_body(m3_ref, v_ref, aux_ref, m4_ref):
    vv = v_ref[...]
    v = vv[:, :HID] + vv[:, HID:]
    acc2 = m3_ref[:, HID:HID + 32] + aux_ref[:, HID:HID + 1] * v[:, :32]
    rs = aux_ref[:, HID + 1:HID + 2] * v[:, 32:]
    m4_ref[...] = jnp.concatenate(
        [rs, acc2, jnp.zeros((rs.shape[0], HID), jnp.float32)], axis=1)


def _t5_body(m4_ref, w_ref, aux_ref,
             cw1_ref, cb1_ref, cw2_ref, cb2_ref, coords_ref):
    wv = w_ref[:, :32] + w_ref[:, 32:64]
    h2 = jax.nn.relu(m4_ref[:, 32:64] + aux_ref[:, HID:HID + 1] * wv)
    t = jax.nn.relu(jnp.dot(h2, cw1_ref[...],
                            preferred_element_type=jnp.float32) + cb1_ref[...])
    coords_ref[...] = jnp.dot(t, cw2_ref[...],
                              preferred_element_type=jnp.float32) + cb2_ref[...]


_BN = 2000


def _tc_call(body, out_shapes, *args):
    in_specs, arrays = [], []
    for a in args:
        if isinstance(a, tuple):
            a, (ncols, cb) = a
            in_specs.append(pl.BlockSpec((_BN, ncols), lambda i, cb=cb: (i, cb)))
        elif a.shape[0] == N:
            in_specs.append(pl.BlockSpec(
                (_BN,) + a.shape[1:],
                lambda i, nd=a.ndim: (i,) + (0,) * (nd - 1)))
        else:
            in_specs.append(pl.BlockSpec(
                a.shape, lambda i, nd=a.ndim: (0,) * nd))
        arrays.append(a)
    out_specs = [
        pl.BlockSpec((_BN,) + s[1:], lambda i, nd=len(s): (i,) + (0,) * (nd - 1))
        for s in out_shapes
    ]
    return pl.pallas_call(
        body,
        grid=(N // _BN,),
        in_specs=in_specs,
        out_specs=out_specs,
        out_shape=[jax.ShapeDtypeStruct(s, jnp.float32) for s in out_shapes],
    )(*arrays)


def kernel(x, conv1_w0, conv1_w1, conv1_w2, conv1_b,
           conv2_w0, conv2_w1, conv2_w2, conv2_b,
           proj_w1, proj_b1, proj_w2, proj_b2,
           coord_w1, coord_b1, coord_w2, coord_b2,
           edge_index):
    er = edge_index.reshape(2, NCHUNK, CW)

    (dp,) = _deg(edge_index.reshape(2, DNCHUNK, DCW))

    aux, a12 = _tc_call(
        _t1_body, [(N, 128), (N, 128)],
        x, conv1_w0, conv1_w1, conv1_w2, conv1_b.reshape(1, HID), dp)

    (u12,) = _prop_s1(a12.reshape(2 * N, HID), er)

    (m2,) = _tc_call(_t2_body, [(N, 128)], aux, u12)

    (q,) = _prop_s2(m2.reshape(2 * N, HID), er)

    (m3,) = _tc_call(
        _t3_body, [(N, 128)],
        m2, q, aux, conv2_w0, conv2_w1, conv2_w2, conv2_b.reshape(1, 32))

    (v,) = _prop_s3(m3.reshape(2 * N, HID), er)

    (m4,) = _tc_call(_t4_body, [(N, 128)], m3, v, aux)

    (w,) = _prop_s4(m4.reshape(4 * N, 32), er)

    (coords,) = _tc_call(
        _t5_body, [(N, 2)],
        m4, w, aux,
        coord_w1, coord_b1.reshape(1, 16), coord_w2, coord_b2.reshape(1, 2))

    return coords
